# initial kernel scaffold (unmeasured)
import jax
import jax.numpy as jnp
from jax import lax
from jax.experimental import pallas as pl
from jax.experimental.pallas import tpu as pltpu

N_DEV = 4
N_COL_TILES = 4


def kernel(x, w_mat):
    m_glob, k_per = x.shape
    _, n = w_mat.shape
    m_per = m_glob // N_DEV
    n_t = n // N_COL_TILES

    def body(x_ref, w_ref, out_ref, comm_ref, send_sems, recv_sems,
             amax_ref, amax_send_sems, amax_recv_sems):
        my = lax.axis_index("i")
        left = lax.rem(my + N_DEV - 1, N_DEV)
        right = lax.rem(my + 1, N_DEV)

        barrier_sem = pltpu.get_barrier_semaphore()
        for nbr in (left, right):
            pl.semaphore_signal(barrier_sem, inc=1, device_id=(nbr,),
                                device_id_type=pl.DeviceIdType.MESH)
        pl.semaphore_wait(barrier_sem, 2)

        def x_chunk(c):
            return x_ref[pl.ds(c * m_per, m_per), :]

        c0 = lax.rem(my + N_DEV - 1, N_DEV)
        xc = x_chunk(c0)
        for t in range(N_COL_TILES):
            cs = pl.ds(t * n_t, n_t)
            comm_ref[0, :, cs] = jnp.dot(
                xc, w_ref[:, cs], preferred_element_type=jnp.float32
            ).astype(jnp.bfloat16)

        for s in range(N_DEV - 1):
            send_slot = s % 2
            recv_slot = (s + 1) % 2
            rdma = pltpu.make_async_remote_copy(
                src_ref=comm_ref.at[send_slot],
                dst_ref=comm_ref.at[recv_slot],
                send_sem=send_sems.at[send_slot],
                recv_sem=recv_sems.at[recv_slot],
                device_id=(right,),
                device_id_type=pl.DeviceIdType.MESH,
            )
            rdma.start()
            rdma.wait()

            c = lax.rem(my + 2 * N_DEV - 2 - s, N_DEV)
            xc = x_chunk(c)
            for t in range(N_COL_TILES):
                cs = pl.ds(t * n_t, n_t)
                acc = jnp.dot(
                    xc, w_ref[:, cs], preferred_element_type=jnp.float32
                ) + comm_ref[recv_slot, :, cs].astype(jnp.float32)
                if s < N_DEV - 2:
                    comm_ref[recv_slot, :, cs] = acc.astype(jnp.bfloat16)
                else:
                    out_ref[:, cs] = jnp.maximum(acc, 0.0)

        amax_local = jnp.max(out_ref[:, :])
        amax_ref[0, :] = jnp.full((128,), amax_local, jnp.float32)
        amax_rdmas = []
        for d in range(1, N_DEV):
            target = lax.rem(my + 2 * N_DEV - d, N_DEV)
            r = pltpu.make_async_remote_copy(
                src_ref=amax_ref.at[0],
                dst_ref=amax_ref.at[d],
                send_sem=amax_send_sems.at[d - 1],
                recv_sem=amax_recv_sems.at[d],
                device_id=(target,),
                device_id_type=pl.DeviceIdType.MESH,
            )
            r.start()
            amax_rdmas.append(r)
        for r in amax_rdmas:
            r.wait_send()
            r.wait_recv()

        g = jnp.max(amax_ref[:, 0])
        scale = g * (1.0 / 448.0)
        inv = 448.0 / g

        for t in range(N_COL_TILES):
            cs = pl.ds(t * n_t, n_t)
            y = out_ref[:, cs]
            q = jnp.minimum(y * inv, 448.0).astype(jnp.float8_e4m3fn)
            out_ref[:, cs] = q.astype(jnp.float32) * scale

    return pl.pallas_call(
        body,
        out_shape=jax.ShapeDtypeStruct((m_per, n), jnp.float32),
        in_specs=[
            pl.BlockSpec(memory_space=pltpu.VMEM),
            pl.BlockSpec(memory_space=pltpu.VMEM),
        ],
        out_specs=pl.BlockSpec(memory_space=pltpu.VMEM),
        scratch_shapes=[
            pltpu.VMEM((2, m_per, n), jnp.bfloat16),
            pltpu.SemaphoreType.DMA((2,)),
            pltpu.SemaphoreType.DMA((2,)),
            pltpu.VMEM((N_DEV, 128), jnp.float32),
            pltpu.SemaphoreType.DMA((N_DEV - 1,)),
            pltpu.SemaphoreType.DMA((N_DEV,)),
        ],
        compiler_params=pltpu.CompilerParams(collective_id=0),
    )(x, w_mat)


# baseline (device time: 519040 ns/iter reference)
import jax
import jax.numpy as jnp
from jax import lax
from jax.experimental import pallas as pl
from jax.experimental.pallas import tpu as pltpu

N_DEV = 4
M = 1024
T = 8
W = 1024


def _rs_call(x, w):
    m_glob, k = x.shape
    n = w.shape[1]

    def body(x_ref, w_ref, y_ref, amax_ref,
             send_buf, recv_buf, own_buf, send_sems, recv_sems):
        t = pl.program_id(0)
        my = lax.axis_index("i")

        @pl.when(t == 0)
        def _():
            bar = pltpu.get_barrier_semaphore()
            for r in range(1, N_DEV):
                pl.semaphore_signal(
                    bar, inc=1,
                    device_id=(lax.rem(my + r, N_DEV),),
                    device_id_type=pl.DeviceIdType.MESH,
                )
            pl.semaphore_wait(bar, N_DEV - 1)

        def mk(src_slot, par, dst_slot, dev):
            return pltpu.make_async_remote_copy(
                src_ref=send_buf.at[src_slot],
                dst_ref=recv_buf.at[par, dst_slot],
                send_sem=send_sems.at[src_slot],
                recv_sem=recv_sems.at[par, dst_slot],
                device_id=(dev,),
                device_id_type=pl.DeviceIdType.MESH,
            )

        @pl.when(t > 0)
        def _():
            for r in range(1, N_DEV):
                mk(r - 1, 0, 3 - r, my).wait_send()

        @pl.when(t > 0)
        def _():
            par = (t - 1) % 2
            for j in range(N_DEV - 1):
                mk(0, par, j, my).wait_recv()
            acc = own_buf[par].astype(jnp.float32)
            for j in range(N_DEV - 1):
                acc = acc + recv_buf[par, j].astype(jnp.float32)
            y = jnp.maximum(acc, 0.0)
            y_ref[...] = y
            tmax = jnp.max(y)

            @pl.when(t == 1)
            def _():
                amax_ref[...] = jnp.full((1, 128), tmax, jnp.float32)

            @pl.when(t > 1)
            def _():
                amax_ref[...] = jnp.maximum(amax_ref[...], tmax)

        @pl.when(t < T)
        def _():
            par = t % 2
            for r in range(1, N_DEV):
                d = lax.rem(my + r, N_DEV)
                xc = x_ref[pl.ds(d * M, M), :]
                send_buf[r - 1] = jnp.dot(
                    xc, w_ref[...], preferred_element_type=jnp.float32
                ).astype(jnp.bfloat16)
                mk(r - 1, par, 3 - r, d).start()
            own_buf[par] = jnp.dot(
                x_ref[pl.ds(my * M, M), :], w_ref[...],
                preferred_element_type=jnp.float32,
            ).astype(jnp.bfloat16)

    return pl.pallas_call(
        body,
        grid=(T + 1,),
        out_shape=[
            jax.ShapeDtypeStruct((M, n), jnp.float32),
            jax.ShapeDtypeStruct((1, 128), jnp.float32),
        ],
        in_specs=[
            pl.BlockSpec((m_glob, k), lambda t: (0, 0),
                         memory_space=pltpu.VMEM),
            pl.BlockSpec((k, W), lambda t: (0, jnp.minimum(t, T - 1)),
                         memory_space=pltpu.VMEM),
        ],
        out_specs=[
            pl.BlockSpec((M, W), lambda t: (0, jnp.maximum(t - 1, 0)),
                         memory_space=pltpu.VMEM),
            pl.BlockSpec((1, 128), lambda t: (0, 0),
                         memory_space=pltpu.VMEM),
        ],
        scratch_shapes=[
            pltpu.VMEM((N_DEV - 1, M, W), jnp.bfloat16),
            pltpu.VMEM((2, N_DEV - 1, M, W), jnp.bfloat16),
            pltpu.VMEM((2, M, W), jnp.bfloat16),
            pltpu.SemaphoreType.DMA((N_DEV - 1,)),
            pltpu.SemaphoreType.DMA((2, N_DEV - 1)),
        ],
        compiler_params=pltpu.CompilerParams(
            collective_id=0, dimension_semantics=("arbitrary",),
            vmem_limit_bytes=63 * 1024 * 1024,
        ),
    )(x, w)


def _quant_call(y, amax_local):
    m, n = y.shape
    t2 = 4
    w2 = n // t2

    def body(y_ref, amax_ref, out_ref, exch, send_sems, recv_sems):
        t = pl.program_id(0)
        my = lax.axis_index("i")

        @pl.when(t == 0)
        def _():
            exch[N_DEV - 1, :] = amax_ref[0, :]
            rdmas = []
            for r in range(1, N_DEV):
                rd = pltpu.make_async_remote_copy(
                    src_ref=exch.at[N_DEV - 1],
                    dst_ref=exch.at[N_DEV - 1 - r],
                    send_sem=send_sems.at[r - 1],
                    recv_sem=recv_sems.at[N_DEV - 1 - r],
                    device_id=(lax.rem(my + r, N_DEV),),
                    device_id_type=pl.DeviceIdType.MESH,
                )
                rd.start()
                rdmas.append(rd)
            for rd in rdmas:
                rd.wait_send()
                rd.wait_recv()

        g = jnp.max(exch[:, 0])
        inv = 448.0 / g
        scale = g * (1.0 / 448.0)
        yv = y_ref[...]
        q = jnp.minimum(yv * inv, 448.0).astype(jnp.float8_e4m3fn)
        out_ref[...] = q.astype(jnp.float32) * scale

    return pl.pallas_call(
        body,
        grid=(t2,),
        out_shape=jax.ShapeDtypeStruct((m, n), jnp.float32),
        in_specs=[
            pl.BlockSpec((m, w2), lambda t: (0, t), memory_space=pltpu.VMEM),
            pl.BlockSpec((1, 128), lambda t: (0, 0),
                         memory_space=pltpu.VMEM),
        ],
        out_specs=pl.BlockSpec((m, w2), lambda t: (0, t),
                               memory_space=pltpu.VMEM),
        scratch_shapes=[
            pltpu.VMEM((N_DEV, 128), jnp.float32),
            pltpu.SemaphoreType.DMA((N_DEV - 1,)),
            pltpu.SemaphoreType.DMA((N_DEV - 1,)),
        ],
        compiler_params=pltpu.CompilerParams(
            dimension_semantics=("arbitrary",),
            vmem_limit_bytes=63 * 1024 * 1024,
        ),
    )(y, amax_local)


def kernel(x, w_mat):
    xb = x.astype(jnp.bfloat16)
    wb = w_mat.astype(jnp.bfloat16)
    y, amax_local = _rs_call(xb, wb)
    return _quant_call(y, amax_local)


# device time: 380876 ns/iter; 1.3628x vs baseline; 1.3628x over previous
import jax
import jax.numpy as jnp
from jax import lax
from jax.experimental import pallas as pl
from jax.experimental.pallas import tpu as pltpu

N_DEV = 4
M = 1024
T = 8
W = 1024
H = W // 2


def _rs_call(x, w):
    m_glob, k = x.shape
    n = w.shape[1]

    def body(x_ref, w_ref, y_ref, amax_ref,
             nbr, own, relay_snd, comb_snd, direct_r, relay_r, comb_r,
             snd_sems, rcv_direct, rcv_relay, rcv_comb):
        t = pl.program_id(0)
        my = lax.axis_index("i")
        dev = [lax.rem(my + 1, N_DEV), lax.rem(my + N_DEV - 1, N_DEV)]
        cols = [slice(0, H), slice(H, W)]

        @pl.when(t == 0)
        def _():
            bar = pltpu.get_barrier_semaphore()
            for s in range(2):
                pl.semaphore_signal(bar, inc=1, device_id=(dev[s],),
                                    device_id_type=pl.DeviceIdType.MESH)
            pl.semaphore_wait(bar, 2)

        def mk_direct(par, s, d):
            return pltpu.make_async_remote_copy(
                src_ref=nbr.at[s, :, cols[s]],
                dst_ref=direct_r.at[par, s],
                send_sem=snd_sems.at[0, s],
                recv_sem=rcv_direct.at[par, s],
                device_id=(d,), device_id_type=pl.DeviceIdType.MESH)

        def mk_relay(par_snd, par, s, d):
            return pltpu.make_async_remote_copy(
                src_ref=relay_snd.at[par_snd, s],
                dst_ref=relay_r.at[par, s],
                send_sem=snd_sems.at[1, s],
                recv_sem=rcv_relay.at[par, s],
                device_id=(d,), device_id_type=pl.DeviceIdType.MESH)

        def mk_comb(par_snd, par, s, d):
            return pltpu.make_async_remote_copy(
                src_ref=comb_snd.at[par_snd, s],
                dst_ref=comb_r.at[par, s],
                send_sem=snd_sems.at[2, s],
                recv_sem=rcv_comb.at[par, s],
                device_id=(d,), device_id_type=pl.DeviceIdType.MESH)

        @pl.when(jnp.logical_and(t >= 1, t <= T))
        def _():
            for s in range(2):
                mk_direct(0, s, my).wait_send()
                mk_relay(lax.rem(t - 1, 2), 0, s, my).wait_send()

        @pl.when(jnp.logical_and(t >= 2, t <= T + 1))
        def _():
            for s in range(2):
                mk_comb(lax.rem(t - 2, 2), 0, s, my).wait_send()

        @pl.when(jnp.logical_and(t >= 1, t <= T))
        def _():
            pr = lax.rem(t - 1, 3)
            pc = lax.rem(t - 1, 2)
            p4 = lax.rem(t - 1, 4)
            for s in range(2):
                mk_relay(0, pr, s, my).wait_recv()
                comb_snd[pc, s] = (
                    nbr[1 - s, :, cols[s]].astype(jnp.float32)
                    + relay_r[pr, s].astype(jnp.float32)
                ).astype(jnp.bfloat16)
                mk_comb(pc, p4, s, dev[1 - s]).start()

        @pl.when(t <= T - 1)
        def _():
            p4 = lax.rem(t, 4)
            p3 = lax.rem(t, 3)
            p2 = lax.rem(t, 2)
            nbr[0] = jnp.dot(
                x_ref[pl.ds(dev[0] * M, M), :], w_ref[...],
                preferred_element_type=jnp.float32).astype(jnp.bfloat16)
            nbr[1] = jnp.dot(
                x_ref[pl.ds(dev[1] * M, M), :], w_ref[...],
                preferred_element_type=jnp.float32).astype(jnp.bfloat16)
            diag = lax.rem(my + 2, N_DEV)
            for s in range(2):
                mk_direct(p4, s, dev[s]).start()
                relay_snd[p2, s] = jnp.dot(
                    x_ref[pl.ds(diag * M, M), :], w_ref[:, cols[s]],
                    preferred_element_type=jnp.float32).astype(jnp.bfloat16)
                mk_relay(p2, p3, s, dev[1 - s]).start()

        @pl.when(t >= 2)
        def _():
            q = lax.rem(t - 2, 4)
            po = lax.rem(t - 2, 2)
            tmax = jnp.float32(0)
            for s in range(2):
                mk_direct(q, s, my).wait_recv()
                mk_comb(0, q, s, my).wait_recv()
                acc = (own[po, :, cols[s]].astype(jnp.float32)
                       + direct_r[q, s].astype(jnp.float32)
                       + comb_r[q, s].astype(jnp.float32))
                ys = jnp.maximum(acc, 0.0)
                y_ref[:, cols[s]] = ys
                tmax = jnp.maximum(tmax, jnp.max(ys))

            @pl.when(t == 2)
            def _():
                amax_ref[...] = jnp.full((1, 128), tmax, jnp.float32)

            @pl.when(t > 2)
            def _():
                amax_ref[...] = jnp.maximum(amax_ref[...], tmax)

        @pl.when(t <= T - 1)
        def _():
            own[lax.rem(t, 2)] = jnp.dot(
                x_ref[pl.ds(my * M, M), :], w_ref[...],
                preferred_element_type=jnp.float32).astype(jnp.bfloat16)

    return pl.pallas_call(
        body,
        grid=(T + 2,),
        out_shape=[
            jax.ShapeDtypeStruct((M, n), jnp.float32),
            jax.ShapeDtypeStruct((1, 128), jnp.float32),
        ],
        in_specs=[
            pl.BlockSpec((m_glob, k), lambda t: (0, 0),
                         memory_space=pltpu.VMEM),
            pl.BlockSpec((k, W), lambda t: (0, jnp.minimum(t, T - 1)),
                         memory_space=pltpu.VMEM),
        ],
        out_specs=[
            pl.BlockSpec((M, W), lambda t: (0, jnp.maximum(t - 2, 0)),
                         memory_space=pltpu.VMEM),
            pl.BlockSpec((1, 128), lambda t: (0, 0),
                         memory_space=pltpu.VMEM),
        ],
        scratch_shapes=[
            pltpu.VMEM((2, M, W), jnp.bfloat16),
            pltpu.VMEM((2, M, W), jnp.bfloat16),
            pltpu.VMEM((2, 2, M, H), jnp.bfloat16),
            pltpu.VMEM((2, 2, M, H), jnp.bfloat16),
            pltpu.VMEM((4, 2, M, H), jnp.bfloat16),
            pltpu.VMEM((3, 2, M, H), jnp.bfloat16),
            pltpu.VMEM((4, 2, M, H), jnp.bfloat16),
            pltpu.SemaphoreType.DMA((3, 2)),
            pltpu.SemaphoreType.DMA((4, 2)),
            pltpu.SemaphoreType.DMA((3, 2)),
            pltpu.SemaphoreType.DMA((4, 2)),
        ],
        compiler_params=pltpu.CompilerParams(
            collective_id=0, dimension_semantics=("arbitrary",),
            vmem_limit_bytes=63 * 1024 * 1024,
        ),
    )(x, w)


def _quant_call(y, amax_local):
    m, n = y.shape
    t2 = 4
    w2 = n // t2

    def body(y_ref, amax_ref, out_ref, exch, send_sems, recv_sems):
        t = pl.program_id(0)
        my = lax.axis_index("i")

        @pl.when(t == 0)
        def _():
            exch[N_DEV - 1, :] = amax_ref[0, :]
            rdmas = []
            for r in range(1, N_DEV):
                rd = pltpu.make_async_remote_copy(
                    src_ref=exch.at[N_DEV - 1],
                    dst_ref=exch.at[N_DEV - 1 - r],
                    send_sem=send_sems.at[r - 1],
                    recv_sem=recv_sems.at[N_DEV - 1 - r],
                    device_id=(lax.rem(my + r, N_DEV),),
                    device_id_type=pl.DeviceIdType.MESH,
                )
                rd.start()
                rdmas.append(rd)
            for rd in rdmas:
                rd.wait_send()
                rd.wait_recv()

        g = jnp.max(exch[:, 0])
        inv = 448.0 / g
        scale = g * (1.0 / 448.0)
        yv = y_ref[...]
        q = jnp.minimum(yv * inv, 448.0).astype(jnp.float8_e4m3fn)
        out_ref[...] = q.astype(jnp.float32) * scale

    return pl.pallas_call(
        body,
        grid=(t2,),
        out_shape=jax.ShapeDtypeStruct((m, n), jnp.float32),
        in_specs=[
            pl.BlockSpec((m, w2), lambda t: (0, t), memory_space=pltpu.VMEM),
            pl.BlockSpec((1, 128), lambda t: (0, 0),
                         memory_space=pltpu.VMEM),
        ],
        out_specs=pl.BlockSpec((m, w2), lambda t: (0, t),
                               memory_space=pltpu.VMEM),
        scratch_shapes=[
            pltpu.VMEM((N_DEV, 128), jnp.float32),
            pltpu.SemaphoreType.DMA((N_DEV - 1,)),
            pltpu.SemaphoreType.DMA((N_DEV - 1,)),
        ],
        compiler_params=pltpu.CompilerParams(
            dimension_semantics=("arbitrary",),
            vmem_limit_bytes=63 * 1024 * 1024,
        ),
    )(y, amax_local)


def kernel(x, w_mat):
    xb = x.astype(jnp.bfloat16)
    wb = w_mat.astype(jnp.bfloat16)
    y, amax_local = _rs_call(xb, wb)
    return _quant_call(y, amax_local)
